# Initial kernel scaffold; baseline (speedup 1.0000x reference)
import jax
import jax.numpy as jnp
from jax.experimental import pallas as pl


def kernel(x, pos_table):
    raise NotImplementedError("write your pallas kernel here")


# speedup vs baseline: 1.9310x; 1.9310x over previous
import jax
import jax.numpy as jnp
from jax.experimental import pallas as pl


def _add_pos_kernel(x_ref, t_ref, o_ref):
    o_ref[...] = x_ref[...] + t_ref[...]


def kernel(x, pos_table):
    B, S, D = x.shape
    TS = 512
    grid = (S // TS, B)
    return pl.pallas_call(
        _add_pos_kernel,
        grid=grid,
        in_specs=[
            pl.BlockSpec((1, TS, D), lambda s, b: (b, s, 0)),
            pl.BlockSpec((TS, D), lambda s, b: (s, 0)),
        ],
        out_specs=pl.BlockSpec((1, TS, D), lambda s, b: (b, s, 0)),
        out_shape=jax.ShapeDtypeStruct((B, S, D), x.dtype),
    )(x, pos_table[:S])


# speedup vs baseline: 2.1154x; 1.0955x over previous
import jax
import jax.numpy as jnp
from jax.experimental import pallas as pl


def _add_pos_kernel(x_ref, t_ref, o_ref):
    o_ref[...] = x_ref[...] + t_ref[...]


def kernel(x, pos_table):
    B, S, D = x.shape
    TS = 1024
    grid = (S // TS, B)
    return pl.pallas_call(
        _add_pos_kernel,
        grid=grid,
        in_specs=[
            pl.BlockSpec((1, TS, D), lambda s, b: (b, s, 0)),
            pl.BlockSpec((TS, D), lambda s, b: (s, 0)),
        ],
        out_specs=pl.BlockSpec((1, TS, D), lambda s, b: (b, s, 0)),
        out_shape=jax.ShapeDtypeStruct((B, S, D), x.dtype),
    )(x, pos_table[:S])


# speedup vs baseline: 2.2933x; 1.0841x over previous
import jax
import jax.numpy as jnp
from jax.experimental import pallas as pl


def _add_pos_kernel(x_ref, t_ref, o_ref):
    o_ref[...] = x_ref[...] + t_ref[...]


def kernel(x, pos_table):
    B, S, D = x.shape
    TS = 2048
    grid = (S // TS, B)
    return pl.pallas_call(
        _add_pos_kernel,
        grid=grid,
        in_specs=[
            pl.BlockSpec((1, TS, D), lambda s, b: (b, s, 0)),
            pl.BlockSpec((TS, D), lambda s, b: (s, 0)),
        ],
        out_specs=pl.BlockSpec((1, TS, D), lambda s, b: (b, s, 0)),
        out_shape=jax.ShapeDtypeStruct((B, S, D), x.dtype),
    )(x, pos_table[:S])
